# baseline (speedup 1.0000x reference)
import functools

import jax
import jax.numpy as jnp
from jax.experimental import pallas as pl
from jax.experimental.pallas import tpu as pltpu

D = 8192
D_OUT = 4096
N_FIXED = 8
BLK = 256
NS = 2


def _ffm_kernel(xp_ref, attrs_ref, w0_ref, w1_ref, b_ref, out_ref):
    xs = pltpu.roll(xp_ref[...], N_FIXED, axis=1)
    col = jax.lax.broadcasted_iota(jnp.int32, (1, D), 1)
    inp = jnp.where(col < N_FIXED, attrs_ref[...], xs)
    i = pl.program_id(0)
    for j, w_ref in enumerate((w0_ref, w1_ref)):
        acc = jax.lax.dot_general(
            inp, w_ref[...], (((1,), (1,)), ((), ())),
            preferred_element_type=jnp.float32)
        off = (i * NS + j) * BLK
        out_ref[:, pl.ds(off, BLK)] = acc + b_ref[:, pl.ds(off, BLK)]


@functools.partial(jax.jit, static_argnames=())
def kernel(x, attrs_init, W, b):
    xp = jnp.pad(x, ((0, 0), (0, N_FIXED)))
    b2 = b.reshape(1, D_OUT)
    grid = (D_OUT // (BLK * NS),)
    out = pl.pallas_call(
        _ffm_kernel,
        grid=grid,
        in_specs=[
            pl.BlockSpec((1, D), lambda i: (0, 0)),
            pl.BlockSpec((1, D), lambda i: (0, 0)),
            pl.BlockSpec((BLK, D), lambda i: (2 * i, 0)),
            pl.BlockSpec((BLK, D), lambda i: (2 * i + 1, 0)),
            pl.BlockSpec((1, D_OUT), lambda i: (0, 0)),
        ],
        out_specs=pl.BlockSpec((1, D_OUT), lambda i: (0, 0)),
        out_shape=jax.ShapeDtypeStruct((1, D_OUT), jnp.float32),
        compiler_params=pltpu.CompilerParams(
            allow_input_fusion=[True, False, False, False, False]),
    )(xp, attrs_init, W, W, b2)
    return out


# speedup vs baseline: 1.0739x; 1.0739x over previous
import functools

import jax
import jax.numpy as jnp
from jax.experimental import pallas as pl
from jax.experimental.pallas import tpu as pltpu

D = 8192
D_OUT = 4096
N_FIXED = 8
BLK = 256


def _ffm_kernel(xp_ref, attrs_ref, w_ref, b_ref, out_ref):
    xs = pltpu.roll(xp_ref[...], N_FIXED, axis=1)
    col = jax.lax.broadcasted_iota(jnp.int32, (1, D), 1)
    inp = jnp.where(col < N_FIXED, attrs_ref[...], xs)
    acc = jax.lax.dot_general(
        inp, w_ref[...], (((1,), (1,)), ((), ())),
        preferred_element_type=jnp.float32)
    i = pl.program_id(0)
    out_ref[:, pl.ds(i * BLK, BLK)] = acc + b_ref[:, pl.ds(i * BLK, BLK)]


@functools.partial(jax.jit, static_argnames=())
def kernel(x, attrs_init, W, b):
    xp = jnp.pad(x, ((0, 0), (0, N_FIXED)))
    b2 = b.reshape(1, D_OUT)
    grid = (D_OUT // BLK,)
    out = pl.pallas_call(
        _ffm_kernel,
        grid=grid,
        in_specs=[
            pl.BlockSpec((1, D), lambda i: (0, 0)),
            pl.BlockSpec((1, D), lambda i: (0, 0)),
            pl.BlockSpec((BLK, D), lambda i: (i, 0)),
            pl.BlockSpec((1, D_OUT), lambda i: (0, 0)),
        ],
        out_specs=pl.BlockSpec((1, D_OUT), lambda i: (0, 0)),
        out_shape=jax.ShapeDtypeStruct((1, D_OUT), jnp.float32),
        compiler_params=pltpu.CompilerParams(
            allow_input_fusion=[True, False, False, False]),
    )(xp, attrs_init, W, b2)
    return out
